# baseline (speedup 1.0000x reference)
import functools

import jax
import jax.numpy as jnp
from jax import lax
from jax.experimental import pallas as pl
from jax.experimental.pallas import tpu as pltpu
from jax.experimental.pallas import tpu_sc as plsc

B = 2
N = 4096
M = 4096
R = 32
K = 16
CD = 128
EH = 128
NK = N * K
NCELL = R * R * R
EPS = 1e-5
SLOPE = 0.05
NW = 32

_f32 = jnp.float32
_i32 = jnp.int32


def _gather_rows(table, idx, chunk):
    bidx = idx.shape[0]
    row_shape = table.shape[1:]
    per_w = bidx // NW
    steps = per_w // chunk
    assert per_w % chunk == 0 and bidx % (8 * NW) == 0

    mesh = plsc.VectorSubcoreMesh(core_axis_name="c", subcore_axis_name="s")

    @functools.partial(
        pl.kernel,
        mesh=mesh,
        out_type=jax.ShapeDtypeStruct((bidx,) + row_shape, table.dtype),
        scratch_types=[
            pltpu.VMEM((chunk,), _i32),
            pltpu.VMEM((chunk,) + row_shape, table.dtype),
            pltpu.SemaphoreType.DMA,
        ],
    )
    def gk(table_hbm, idx_hbm, out_hbm, idx_v, rows_v, sem):
        wid = lax.axis_index("s") * 2 + lax.axis_index("c")
        base = wid * per_w

        @pl.loop(0, steps)
        def _(i):
            off = base + i * chunk
            pltpu.sync_copy(idx_hbm.at[pl.ds(off, chunk)], idx_v)
            pltpu.async_copy(table_hbm.at[idx_v], rows_v, sem).wait()
            pltpu.sync_copy(rows_v, out_hbm.at[pl.ds(off, chunk)])

    return gk(table, idx)


def _leaky(y):
    return jnp.where(y > 0, y, SLOPE * y)


def _dot(a, b):
    return jnp.dot(a, b, preferred_element_type=_f32)


def _corners_2d(xyz, boff):
    c = jnp.clip(xyz[:, 0:3], 0.0, 1.0) * (R - 1.0)
    c0 = jnp.minimum(jnp.floor(c), R - 2.0)
    f = c - c0
    c0i = c0.astype(_i32)
    x0, y0, z0 = c0i[:, 0:1], c0i[:, 1:2], c0i[:, 2:3]
    fx, fy, fz = f[:, 0:1], f[:, 1:2], f[:, 2:3]
    idxs = []
    ws = []
    for dx in (0, 1):
        for dy in (0, 1):
            xi = x0 + dx
            yi = y0 + dy
            idxs.append((xi * R + yi) * R + z0 + boff)
            wxy = (fx if dx else 1.0 - fx) * (fy if dy else 1.0 - fy)
            ws.append(wxy * (1.0 - fz))
            ws.append(wxy * fz)
    return (
        jnp.concatenate(idxs, axis=1),
        jnp.concatenate(ws, axis=1),
    )


_TCOL = 2048


def _grid_transpose(feat_out):
    def body(x_ref, o_ref):
        cur = x_ref[0].T
        shifted = jnp.concatenate([cur[1:, :], cur[0:1, :]], axis=0)
        o_ref[:, 0:CD] = cur.astype(jnp.bfloat16)
        o_ref[:, CD:] = shifted.astype(jnp.bfloat16)

    nblk = NCELL // _TCOL
    return pl.pallas_call(
        body,
        grid=(B, nblk),
        in_specs=[pl.BlockSpec((1, CD, _TCOL), lambda b, j: (b, 0, j))],
        out_specs=pl.BlockSpec((_TCOL, 2 * CD), lambda b, j: (b * nblk + j, 0)),
        out_shape=jax.ShapeDtypeStruct((B * NCELL, 2 * CD), jnp.bfloat16),
    )(feat_out)


def _point_cba(feat_pt, inputs, w, bb, g, be):
    def body(x_ref, in_ref, w_ref, p_ref, o_ref):
        wt = w_ref[...].T
        n = 2.0 * M
        ys = []
        s1 = jnp.zeros((1, CD), _f32)
        s2 = jnp.zeros((1, CD), _f32)
        for b in range(B):
            y = _dot(x_ref[b].T, wt) + p_ref[0:1, :]
            s1 = s1 + jnp.sum(y, axis=0, keepdims=True)
            s2 = s2 + jnp.sum(y * y, axis=0, keepdims=True)
            ys.append(y)
        mu = s1 / n
        var = s2 / n - mu * mu
        sc = p_ref[1:2, :] * lax.rsqrt(var + EPS)
        sh = p_ref[2:3, :] - mu * sc
        for b in range(B):
            o_ref[b * M : (b + 1) * M, 0:CD] = _leaky(ys[b] * sc + sh)
            o_ref[b * M : (b + 1) * M, CD : 2 * CD] = jnp.concatenate(
                [in_ref[b], jnp.zeros((M, CD - 3), _f32)], axis=1
            )

    pvec = jnp.stack([bb, g, be], axis=0)
    return pl.pallas_call(
        body,
        grid=(1,),
        in_specs=[
            pl.BlockSpec((B, EH, M), lambda i: (0, 0, 0)),
            pl.BlockSpec((B, M, 3), lambda i: (0, 0, 0)),
            pl.BlockSpec((CD, EH), lambda i: (0, 0)),
            pl.BlockSpec((3, CD), lambda i: (0, 0)),
        ],
        out_specs=pl.BlockSpec((B * M, 2 * CD), lambda i: (0, 0)),
        out_shape=jax.ShapeDtypeStruct((B * M, 2 * CD), _f32),
    )(feat_pt, inputs, w, pvec)


def _prep(p):
    def body(p_ref, qc_ref, qw_ref):
        for b in range(B):
            ci, wc = _corners_2d(p_ref[b], b * NCELL)
            qc_ref[b * N : (b + 1) * N, :] = ci
            qw_ref[b * N : (b + 1) * N, :] = wc

    return pl.pallas_call(
        body,
        grid=(1,),
        in_specs=[
            pl.BlockSpec((B, N, 3), lambda i: (0, 0, 0)),
        ],
        out_specs=[
            pl.BlockSpec((B * N, 4), lambda i: (0, 0)),
            pl.BlockSpec((B * N, 8), lambda i: (0, 0)),
        ],
        out_shape=[
            jax.ShapeDtypeStruct((B * N, 4), _i32),
            jax.ShapeDtypeStruct((B * N, 8), _f32),
        ],
    )(p)


_TQ = 256


def _knn(p, inputs):
    def body(q_ref, db_ref, o_ref):
        b = pl.program_id(0)
        q = q_ref[0]
        db = db_ref[0]
        qq = jnp.sum(q * q, axis=1, keepdims=True)
        dd = jnp.sum(db * db, axis=1, keepdims=True)
        cross = _dot(q, db.T)
        d2 = qq + dd.T - 2.0 * cross
        iota = lax.broadcasted_iota(_i32, (_TQ, M), 1)
        big = jnp.float32(3.4e38)
        cols = []
        for _ in range(K):
            mval = jnp.min(d2, axis=1, keepdims=True)
            amin = jnp.min(
                jnp.where(d2 == mval, iota, M), axis=1, keepdims=True
            )
            cols.append(amin)
            d2 = jnp.where(iota == amin, big, d2)
        o_ref[0] = jnp.concatenate(cols, axis=1) + b * M

    return pl.pallas_call(
        body,
        grid=(B, N // _TQ),
        in_specs=[
            pl.BlockSpec((1, _TQ, 3), lambda b, i: (b, i, 0)),
            pl.BlockSpec((1, M, 3), lambda b, i: (b, 0, 0)),
        ],
        out_specs=pl.BlockSpec((1, _TQ, K), lambda b, i: (b, i, 0)),
        out_shape=jax.ShapeDtypeStruct((B, N, K), _i32),
    )(p, inputs)


_TN_EX = 256


def _extract(pfx, p_exp):
    nsteps = N // _TN_EX

    def body(rows_ref, p_ref, ci_ref, w_ref, st_ref):
        b = pl.program_id(0)
        step = pl.program_id(1)

        @pl.when(jnp.logical_and(b == 0, step == 0))
        def _():
            st_ref[...] = jnp.zeros_like(st_ref)

        xyz = rows_ref[:, CD : CD + 3]
        ci, wc = _corners_2d(xyz, b * NCELL)
        ci_ref[...] = ci
        w_ref[...] = wc
        pd = p_ref[...] - xyz
        px, py, pz = pd[:, 0:1], pd[:, 1:2], pd[:, 2:3]
        vals = [
            px, py, pz,
            px * px, py * py, pz * pz,
            px * py, px * pz, py * pz,
        ]
        acc = jnp.concatenate([jnp.sum(v, axis=0, keepdims=True) for v in vals]
                              + [jnp.zeros((1, 7), _f32)], axis=1)
        st_ref[...] += acc

    return pl.pallas_call(
        body,
        grid=(B, nsteps),
        in_specs=[
            pl.BlockSpec((_TN_EX * K, 2 * CD), lambda b, i: (b * nsteps + i, 0)),
            pl.BlockSpec((_TN_EX * K, 3), lambda b, i: (b * nsteps + i, 0)),
        ],
        out_specs=[
            pl.BlockSpec((_TN_EX * K, 4), lambda b, i: (b * nsteps + i, 0)),
            pl.BlockSpec((_TN_EX * K, 8), lambda b, i: (b * nsteps + i, 0)),
            pl.BlockSpec((1, 16), lambda b, i: (0, 0)),
        ],
        out_shape=[
            jax.ShapeDtypeStruct((B * NK, 4), _i32),
            jax.ShapeDtypeStruct((B * NK, 8), _f32),
            jax.ShapeDtypeStruct((1, 16), _f32),
        ],
    )(pfx, p_exp)


_TN_I = 1024


def _interp1(qrows, qw, w1, b1):
    nsteps = (B * N) // _TN_I

    def body(rows_ref, qw_ref, w1_ref, b1_ref, y_ref, ys_ref):
        @pl.when(pl.program_id(0) == 0)
        def _():
            ys_ref[...] = jnp.zeros_like(ys_ref)

        rows = rows_ref[...].astype(_f32).reshape(_TN_I, 8, CD)
        qwv = qw_ref[...]
        acc = rows[:, 0, :] * qwv[:, 0:1]
        for c in range(1, 8):
            acc = acc + rows[:, c, :] * qwv[:, c : c + 1]
        y = _dot(acc, w1_ref[...].T) + b1_ref[0:1, :]
        y_ref[...] = y
        ys_ref[0:1, :] += jnp.sum(y, axis=0, keepdims=True)
        ys_ref[1:2, :] += jnp.sum(y * y, axis=0, keepdims=True)

    return pl.pallas_call(
        body,
        grid=(nsteps,),
        in_specs=[
            pl.BlockSpec((_TN_I * 8, CD), lambda i: (i, 0)),
            pl.BlockSpec((_TN_I, 8), lambda i: (i, 0)),
            pl.BlockSpec((CD, CD), lambda i: (0, 0)),
            pl.BlockSpec((1, CD), lambda i: (0, 0)),
        ],
        out_specs=[
            pl.BlockSpec((_TN_I, CD), lambda i: (i, 0)),
            pl.BlockSpec((2, CD), lambda i: (0, 0)),
        ],
        out_shape=[
            jax.ShapeDtypeStruct((B * N, CD), _f32),
            jax.ShapeDtypeStruct((2, CD), _f32),
        ],
    )(qrows, qw, w1, b1)


_TN_A = 64


def _pass_a(crows, wcorn, pfx, p_exp, pdstats, wpd, ppd, w1, b1):
    nsteps = N // _TN_A
    rows = _TN_A * K

    def body(cr_ref, wc_ref, pf_ref, p_ref, st_ref, wpd_ref,
             ppd_ref, w1a_ref, w1b_ref, w1c_ref, b1_ref, y_ref, ys_ref):
        b = pl.program_id(0)
        step = pl.program_id(1)

        @pl.when(jnp.logical_and(b == 0, step == 0))
        def _():
            ys_ref[...] = jnp.zeros_like(ys_ref)

        cr = cr_ref[...].astype(_f32).reshape(rows, 8, CD)
        wc = wc_ref[...]
        vfeat = cr[:, 0, :] * wc[:, 0:1]
        for c in range(1, 8):
            vfeat = vfeat + cr[:, c, :] * wc[:, c : c + 1]

        pd = p_ref[...] - pf_ref[:, CD : CD + 3]
        wpd = wpd_ref[...]
        ypd = _dot(pd, wpd.T) + ppd_ref[0:1, :]
        n = float(B * NK)
        st = st_ref[...]
        mx, my, mz = st[0, 0] / n, st[0, 1] / n, st[0, 2] / n
        cxx = st[0, 3] / n - mx * mx
        cyy = st[0, 4] / n - my * my
        czz = st[0, 5] / n - mz * mz
        cxy = st[0, 6] / n - mx * my
        cxz = st[0, 7] / n - mx * mz
        cyz = st[0, 8] / n - my * mz
        wx, wy, wz = wpd[:, 0:1].T, wpd[:, 1:2].T, wpd[:, 2:3].T
        mu = wx * mx + wy * my + wz * mz + ppd_ref[0:1, :]
        var = (
            wx * wx * cxx + wy * wy * cyy + wz * wz * czz
            + 2.0 * (wx * wy * cxy + wx * wz * cxz + wy * wz * cyz)
        )
        sc = ppd_ref[1:2, :] * lax.rsqrt(var + EPS)
        sh = ppd_ref[2:3, :] - mu * sc
        xpd = _leaky(ypd * sc + sh)

        y1 = (
            _dot(pf_ref[:, 0:CD], w1a_ref[...].T)
            + _dot(vfeat, w1b_ref[...].T)
            + _dot(xpd, w1c_ref[...].T)
            + b1_ref[0:1, :]
        )
        y_ref[...] = y1
        ys_ref[0:1, :] += jnp.sum(y1, axis=0, keepdims=True)
        ys_ref[1:2, :] += jnp.sum(y1 * y1, axis=0, keepdims=True)

    return pl.pallas_call(
        body,
        grid=(B, nsteps),
        in_specs=[
            pl.BlockSpec((rows * 8, CD), lambda b, i: (b * nsteps + i, 0)),
            pl.BlockSpec((rows, 8), lambda b, i: (b * nsteps + i, 0)),
            pl.BlockSpec((rows, 2 * CD), lambda b, i: (b * nsteps + i, 0)),
            pl.BlockSpec((rows, 3), lambda b, i: (b * nsteps + i, 0)),
            pl.BlockSpec((1, 16), lambda b, i: (0, 0)),
            pl.BlockSpec((CD, 3), lambda b, i: (0, 0)),
            pl.BlockSpec((3, CD), lambda b, i: (0, 0)),
            pl.BlockSpec((CD, CD), lambda b, i: (0, 0)),
            pl.BlockSpec((CD, CD), lambda b, i: (0, 0)),
            pl.BlockSpec((CD, CD), lambda b, i: (0, 0)),
            pl.BlockSpec((1, CD), lambda b, i: (0, 0)),
        ],
        out_specs=[
            pl.BlockSpec((rows, CD), lambda b, i: (b * nsteps + i, 0)),
            pl.BlockSpec((2, CD), lambda b, i: (0, 0)),
        ],
        out_shape=[
            jax.ShapeDtypeStruct((B * NK, CD), _f32),
            jax.ShapeDtypeStruct((2, CD), _f32),
        ],
    )(crows, wcorn, pfx, p_exp, pdstats,
      wpd, ppd, w1[:, 0:CD], w1[:, CD : 2 * CD], w1[:, 2 * CD :], b1)


_TROW = 2048


def _pass_bn_mm(y_in, ystats, gprev, beprev, w, bnext, nrows, trow=_TROW):
    nsteps = nrows // trow

    def body(y_ref, ys_ref, gb_ref, w_ref, o_ref, os_ref):
        @pl.when(pl.program_id(0) == 0)
        def _():
            os_ref[...] = jnp.zeros_like(os_ref)

        n = float(nrows)
        ys = ys_ref[...]
        mu = ys[0:1, :] / n
        var = ys[1:2, :] / n - mu * mu
        sc = gb_ref[0:1, :] * lax.rsqrt(var + EPS)
        sh = gb_ref[1:2, :] - mu * sc
        x = _leaky(y_ref[...] * sc + sh)
        y2 = _dot(x, w_ref[...].T) + gb_ref[2:3, :]
        o_ref[...] = y2
        os_ref[0:1, :] += jnp.sum(y2, axis=0, keepdims=True)
        os_ref[1:2, :] += jnp.sum(y2 * y2, axis=0, keepdims=True)

    gb = jnp.concatenate(
        [gprev[None, :], beprev[None, :], bnext[None, :]], axis=0
    )
    return pl.pallas_call(
        body,
        grid=(nsteps,),
        in_specs=[
            pl.BlockSpec((trow, CD), lambda i: (i, 0)),
            pl.BlockSpec((2, CD), lambda i: (0, 0)),
            pl.BlockSpec((3, CD), lambda i: (0, 0)),
            pl.BlockSpec((CD, CD), lambda i: (0, 0)),
        ],
        out_specs=[
            pl.BlockSpec((trow, CD), lambda i: (i, 0)),
            pl.BlockSpec((2, CD), lambda i: (0, 0)),
        ],
        out_shape=[
            jax.ShapeDtypeStruct((nrows, CD), _f32),
            jax.ShapeDtypeStruct((2, CD), _f32),
        ],
    )(y_in, ystats, gb, w)


_TN_D = 256


def _pass_d(y2, y2s, y3, y3s, g2, be2, g3, be3, wkw2, bkw2,
            yi2, yi2s, gi2, bei2, wb1, bb1):
    nsteps = N // _TN_D
    rows = _TN_D * K

    def body(y2_ref, y2s_ref, y3_ref, y3s_ref, pv_ref, kw2_ref,
             yi2_ref, yi2s_ref, wb1a_ref, wb1b_ref, y4_ref, y4s_ref):
        b = pl.program_id(0)
        step = pl.program_id(1)

        @pl.when(jnp.logical_and(b == 0, step == 0))
        def _():
            y4s_ref[...] = jnp.zeros_like(y4s_ref)

        n = float(B * NK)
        y2s = y2s_ref[...]
        mu2 = y2s[0:1, :] / n
        var2 = y2s[1:2, :] / n - mu2 * mu2
        sc2 = pv_ref[0:1, :] * lax.rsqrt(var2 + EPS)
        sh2 = pv_ref[1:2, :] - mu2 * sc2
        x2 = _leaky(y2_ref[...] * sc2 + sh2)

        y3s = y3s_ref[...]
        mu3 = y3s[0:1, :] / n
        var3 = y3s[1:2, :] / n - mu3 * mu3
        sc3 = pv_ref[2:3, :] * lax.rsqrt(var3 + EPS)
        sh3 = pv_ref[3:4, :] - mu3 * sc3
        x3 = _leaky(y3_ref[...] * sc3 + sh3)

        s = jnp.sum(x3 * kw2_ref[0:1, :], axis=1, keepdims=True)
        sg = s.reshape(_TN_D, K, 1)
        smax = sg[:, 0, :]
        for k in range(1, K):
            smax = jnp.maximum(smax, sg[:, k, :])
        x2g = x2.reshape(_TN_D, K, CD)
        den = jnp.zeros((_TN_D, 1), _f32)
        vf = jnp.zeros((_TN_D, CD), _f32)
        for k in range(K):
            e = jnp.exp(sg[:, k, :] - smax)
            den = den + e
            vf = vf + x2g[:, k, :] * e
        vf = vf / den

        ni = float(B * N)
        yis = yi2s_ref[...]
        mui = yis[0:1, :] / ni
        vari = yis[1:2, :] / ni - mui * mui
        sci = pv_ref[6:7, :] * lax.rsqrt(vari + EPS)
        shi = pv_ref[7:8, :] - mui * sci
        vfp = _leaky(yi2_ref[...] * sci + shi)

        y4 = (
            _dot(vf, wb1a_ref[...].T)
            + _dot(vfp, wb1b_ref[...].T)
            + pv_ref[5:6, :]
        )
        y4_ref[...] = y4
        y4s_ref[0:1, :] += jnp.sum(y4, axis=0, keepdims=True)
        y4s_ref[1:2, :] += jnp.sum(y4 * y4, axis=0, keepdims=True)

    pvec = jnp.stack(
        [g2, be2, g3, be3,
         jnp.full((CD,), bkw2[0], _f32), bb1, gi2, bei2], axis=0
    )
    return pl.pallas_call(
        body,
        grid=(B, nsteps),
        in_specs=[
            pl.BlockSpec((rows, CD), lambda b, i: (b * nsteps + i, 0)),
            pl.BlockSpec((2, CD), lambda b, i: (0, 0)),
            pl.BlockSpec((rows, CD), lambda b, i: (b * nsteps + i, 0)),
            pl.BlockSpec((2, CD), lambda b, i: (0, 0)),
            pl.BlockSpec((8, CD), lambda b, i: (0, 0)),
            pl.BlockSpec((1, CD), lambda b, i: (0, 0)),
            pl.BlockSpec((_TN_D, CD), lambda b, i: (b * nsteps + i, 0)),
            pl.BlockSpec((2, CD), lambda b, i: (0, 0)),
            pl.BlockSpec((CD, CD), lambda b, i: (0, 0)),
            pl.BlockSpec((CD, CD), lambda b, i: (0, 0)),
        ],
        out_specs=[
            pl.BlockSpec((_TN_D, CD), lambda b, i: (b * nsteps + i, 0)),
            pl.BlockSpec((2, CD), lambda b, i: (0, 0)),
        ],
        out_shape=[
            jax.ShapeDtypeStruct((B * N, CD), _f32),
            jax.ShapeDtypeStruct((2, CD), _f32),
        ],
    )(y2, y2s, y3, y3s, pvec, wkw2, yi2, yi2s, wb1[:, 0:CD], wb1[:, CD:])


def _final(y4, y4s, g, be, wb2, bb2):
    def body(y_ref, ys_ref, pv_ref, w_ref, o_ref):
        n = float(B * N)
        ys = ys_ref[...]
        mu = ys[0:1, :] / n
        var = ys[1:2, :] / n - mu * mu
        sc = pv_ref[0:1, :] * lax.rsqrt(var + EPS)
        sh = pv_ref[1:2, :] - mu * sc
        x = _leaky(y_ref[...] * sc + sh)
        o_ref[...] = (
            jnp.sum(x * w_ref[0:1, :], axis=1, keepdims=True) + pv_ref[2, 0]
        )

    pvec = jnp.stack([g, be, jnp.full((CD,), bb2[0], _f32)], axis=0)
    return pl.pallas_call(
        body,
        grid=(1,),
        in_specs=[
            pl.BlockSpec((B * N, CD), lambda i: (0, 0)),
            pl.BlockSpec((2, CD), lambda i: (0, 0)),
            pl.BlockSpec((3, CD), lambda i: (0, 0)),
            pl.BlockSpec((1, CD), lambda i: (0, 0)),
        ],
        out_specs=pl.BlockSpec((B * N, 1), lambda i: (0, 0)),
        out_shape=jax.ShapeDtypeStruct((B * N, 1), _f32),
    )(y4, y4s, pvec, wb2)


def kernel(p, z, feat_g, feat_out, feat_pt, inputs, params):
    pr = params

    gt_bf = _grid_transpose(feat_out)
    gt = lax.bitcast_convert_type(
        gt_bf.reshape(B * NCELL, CD, 2), _f32
    )
    fptt = _point_cba(
        feat_pt, inputs,
        pr["point_W"], pr["point_b"], pr["point_g"], pr["point_be"]
    )
    qcidx, qw = _prep(p)
    idxs = _knn(p, inputs)

    idx_flat = idxs.reshape(B * NK)
    p_exp = jnp.broadcast_to(p[:, :, None, :], (B, N, K, 3)).reshape(B * NK, 3)

    qrows = _gather_rows(gt, qcidx.reshape(B * N * 4), 512)
    qrows = lax.bitcast_convert_type(qrows, jnp.bfloat16).reshape(
        B * N * 8, CD
    )
    pfx = _gather_rows(fptt, idx_flat, 256)

    cidx, wcorn, pdstats = _extract(pfx, p_exp)
    crows = _gather_rows(gt, cidx.reshape(B * NK * 4), 512)
    crows = lax.bitcast_convert_type(crows, jnp.bfloat16).reshape(
        B * NK * 8, CD
    )

    yi1, yi1s = _interp1(qrows, qw, pr["interp1_W"], pr["interp1_b"][None, :])
    yi2, yi2s = _pass_bn_mm(
        yi1, yi1s, pr["interp1_g"], pr["interp1_be"],
        pr["interp2_W"], pr["interp2_b"], B * N, 1024,
    )

    y1, y1s = _pass_a(
        crows, wcorn, pfx, p_exp, pdstats,
        pr["pdiff_W"],
        jnp.stack([pr["pdiff_b"], pr["pdiff_g"], pr["pdiff_be"]], 0),
        pr["knn1_W"], pr["knn1_b"][None, :],
    )
    y2, y2s = _pass_bn_mm(
        y1, y1s, pr["knn1_g"], pr["knn1_be"], pr["knn2_W"], pr["knn2_b"],
        B * NK,
    )
    y3, y3s = _pass_bn_mm(
        y2, y2s, pr["knn2_g"], pr["knn2_be"], pr["kw1_W"], pr["kw1_b"],
        B * NK,
    )
    y4, y4s = _pass_d(
        y2, y2s, y3, y3s,
        pr["knn2_g"], pr["knn2_be"], pr["kw1_g"], pr["kw1_be"],
        pr["kw2_W"], pr["kw2_b"],
        yi2, yi2s, pr["interp2_g"], pr["interp2_be"],
        pr["block1_W"], pr["block1_b"],
    )
    out = _final(y4, y4s, pr["block1_g"], pr["block1_be"],
                 pr["block2_W"], pr["block2_b"])
    return out.reshape(B, 1, N)


# speedup vs baseline: 22.2240x; 22.2240x over previous
import functools

import jax
import jax.numpy as jnp
from jax import lax
from jax.experimental import pallas as pl
from jax.experimental.pallas import tpu as pltpu
from jax.experimental.pallas import tpu_sc as plsc

B = 2
N = 4096
M = 4096
R = 32
K = 16
CD = 128
EH = 128
NK = N * K
NCELL = R * R * R
EPS = 1e-5
SLOPE = 0.05
NW = 32

_f32 = jnp.float32
_i32 = jnp.int32


def _gather_rows(table, idx, chunk):
    bidx = idx.shape[0]
    row_shape = table.shape[1:]
    per_w = bidx // NW
    steps = per_w // chunk
    assert per_w % chunk == 0 and bidx % (8 * NW) == 0

    mesh = plsc.VectorSubcoreMesh(core_axis_name="c", subcore_axis_name="s")

    @functools.partial(
        pl.kernel,
        mesh=mesh,
        out_type=jax.ShapeDtypeStruct((bidx,) + row_shape, table.dtype),
        scratch_types=[
            pltpu.VMEM((chunk,), _i32),
            pltpu.VMEM((chunk,) + row_shape, table.dtype),
            pltpu.SemaphoreType.DMA,
        ],
    )
    def gk(table_hbm, idx_hbm, out_hbm, idx_v, rows_v, sem):
        wid = lax.axis_index("s") * 2 + lax.axis_index("c")
        base = wid * per_w

        @pl.loop(0, steps)
        def _(i):
            off = base + i * chunk
            pltpu.sync_copy(idx_hbm.at[pl.ds(off, chunk)], idx_v)
            pltpu.async_copy(table_hbm.at[idx_v], rows_v, sem).wait()
            pltpu.sync_copy(rows_v, out_hbm.at[pl.ds(off, chunk)])

    return gk(table, idx)


def _leaky(y):
    return jnp.where(y > 0, y, SLOPE * y)


def _dot(a, b):
    return jnp.dot(a, b, preferred_element_type=_f32)


def _corners_2d(xyz, boff):
    c = jnp.clip(xyz[:, 0:3], 0.0, 1.0) * (R - 1.0)
    c0 = jnp.minimum(jnp.floor(c), R - 2.0)
    f = c - c0
    c0i = c0.astype(_i32)
    x0, y0, z0 = c0i[:, 0:1], c0i[:, 1:2], c0i[:, 2:3]
    fx, fy, fz = f[:, 0:1], f[:, 1:2], f[:, 2:3]
    idxs = []
    ws = []
    for dx in (0, 1):
        for dy in (0, 1):
            xi = x0 + dx
            yi = y0 + dy
            base = (xi * R + yi) * R + z0 + boff
            wxy = (fx if dx else 1.0 - fx) * (fy if dy else 1.0 - fy)
            idxs.append(base)
            idxs.append(base + 1)
            ws.append(wxy * (1.0 - fz))
            ws.append(wxy * fz)
    return (
        jnp.concatenate(idxs, axis=1),
        jnp.concatenate(ws, axis=1),
    )


_TCOL = 2048


def _grid_transpose(feat_out):
    def body(x_ref, o_ref):
        o_ref[...] = x_ref[0].T

    nblk = NCELL // _TCOL
    return pl.pallas_call(
        body,
        grid=(B, nblk),
        in_specs=[pl.BlockSpec((1, CD, _TCOL), lambda b, j: (b, 0, j))],
        out_specs=pl.BlockSpec((_TCOL, CD), lambda b, j: (b * nblk + j, 0)),
        out_shape=jax.ShapeDtypeStruct((B * NCELL, CD), _f32),
    )(feat_out)


def _point_cba(feat_pt, inputs, w, bb, g, be):
    def body(x_ref, in_ref, w_ref, p_ref, o_ref):
        wt = w_ref[...].T
        n = 2.0 * M
        ys = []
        s1 = jnp.zeros((1, CD), _f32)
        s2 = jnp.zeros((1, CD), _f32)
        for b in range(B):
            y = _dot(x_ref[b].T, wt) + p_ref[0:1, :]
            s1 = s1 + jnp.sum(y, axis=0, keepdims=True)
            s2 = s2 + jnp.sum(y * y, axis=0, keepdims=True)
            ys.append(y)
        mu = s1 / n
        var = s2 / n - mu * mu
        sc = p_ref[1:2, :] * lax.rsqrt(var + EPS)
        sh = p_ref[2:3, :] - mu * sc
        for b in range(B):
            o_ref[b * M : (b + 1) * M, 0:CD] = _leaky(ys[b] * sc + sh)
            o_ref[b * M : (b + 1) * M, CD : 2 * CD] = jnp.concatenate(
                [in_ref[b], jnp.zeros((M, CD - 3), _f32)], axis=1
            )

    pvec = jnp.stack([bb, g, be], axis=0)
    return pl.pallas_call(
        body,
        grid=(1,),
        in_specs=[
            pl.BlockSpec((B, EH, M), lambda i: (0, 0, 0)),
            pl.BlockSpec((B, M, 3), lambda i: (0, 0, 0)),
            pl.BlockSpec((CD, EH), lambda i: (0, 0)),
            pl.BlockSpec((3, CD), lambda i: (0, 0)),
        ],
        out_specs=pl.BlockSpec((B * M, 2 * CD), lambda i: (0, 0)),
        out_shape=jax.ShapeDtypeStruct((B * M, 2 * CD), _f32),
    )(feat_pt, inputs, w, pvec)


def _prep(p):
    def body(p_ref, qc_ref, qw_ref):
        for b in range(B):
            ci, wc = _corners_2d(p_ref[b], b * NCELL)
            qc_ref[b * N : (b + 1) * N, :] = ci
            qw_ref[b * N : (b + 1) * N, :] = wc

    return pl.pallas_call(
        body,
        grid=(1,),
        in_specs=[
            pl.BlockSpec((B, N, 3), lambda i: (0, 0, 0)),
        ],
        out_specs=[
            pl.BlockSpec((B * N, 8), lambda i: (0, 0)),
            pl.BlockSpec((B * N, 8), lambda i: (0, 0)),
        ],
        out_shape=[
            jax.ShapeDtypeStruct((B * N, 8), _i32),
            jax.ShapeDtypeStruct((B * N, 8), _f32),
        ],
    )(p)


_TQ = 256


def _knn(p, inputs):
    def body(q_ref, db_ref, o_ref):
        b = pl.program_id(0)
        q = q_ref[0]
        db = db_ref[0]
        qq = jnp.sum(q * q, axis=1, keepdims=True)
        dd = jnp.sum(db * db, axis=1, keepdims=True)
        cross = _dot(q, db.T)
        d2 = qq + dd.T - 2.0 * cross
        iota = lax.broadcasted_iota(_i32, (_TQ, M), 1)
        big = jnp.float32(3.4e38)
        cols = []
        for _ in range(K):
            mval = jnp.min(d2, axis=1, keepdims=True)
            amin = jnp.min(
                jnp.where(d2 == mval, iota, M), axis=1, keepdims=True
            )
            cols.append(amin)
            d2 = jnp.where(iota == amin, big, d2)
        o_ref[0] = jnp.concatenate(cols, axis=1) + b * M

    return pl.pallas_call(
        body,
        grid=(B, N // _TQ),
        in_specs=[
            pl.BlockSpec((1, _TQ, 3), lambda b, i: (b, i, 0)),
            pl.BlockSpec((1, M, 3), lambda b, i: (b, 0, 0)),
        ],
        out_specs=pl.BlockSpec((1, _TQ, K), lambda b, i: (b, i, 0)),
        out_shape=jax.ShapeDtypeStruct((B, N, K), _i32),
    )(p, inputs)


_TN_EX = 256


def _extract(pfx, p_exp):
    nsteps = N // _TN_EX

    def body(rows_ref, p_ref, ci_ref, w_ref, st_ref):
        b = pl.program_id(0)
        step = pl.program_id(1)

        @pl.when(jnp.logical_and(b == 0, step == 0))
        def _():
            st_ref[...] = jnp.zeros_like(st_ref)

        xyz = rows_ref[:, CD : CD + 3]
        ci, wc = _corners_2d(xyz, b * NCELL)
        ci_ref[...] = ci
        w_ref[...] = wc
        pd = p_ref[...] - xyz
        px, py, pz = pd[:, 0:1], pd[:, 1:2], pd[:, 2:3]
        vals = [
            px, py, pz,
            px * px, py * py, pz * pz,
            px * py, px * pz, py * pz,
        ]
        acc = jnp.concatenate([jnp.sum(v, axis=0, keepdims=True) for v in vals]
                              + [jnp.zeros((1, 7), _f32)], axis=1)
        st_ref[...] += acc

    return pl.pallas_call(
        body,
        grid=(B, nsteps),
        in_specs=[
            pl.BlockSpec((_TN_EX * K, 2 * CD), lambda b, i: (b * nsteps + i, 0)),
            pl.BlockSpec((_TN_EX * K, 3), lambda b, i: (b * nsteps + i, 0)),
        ],
        out_specs=[
            pl.BlockSpec((_TN_EX * K, 8), lambda b, i: (b * nsteps + i, 0)),
            pl.BlockSpec((_TN_EX * K, 8), lambda b, i: (b * nsteps + i, 0)),
            pl.BlockSpec((1, 16), lambda b, i: (0, 0)),
        ],
        out_shape=[
            jax.ShapeDtypeStruct((B * NK, 8), _i32),
            jax.ShapeDtypeStruct((B * NK, 8), _f32),
            jax.ShapeDtypeStruct((1, 16), _f32),
        ],
    )(pfx, p_exp)


_TN_I = 1024


def _interp1(qrows, qw, w1, b1):
    nsteps = (B * N) // _TN_I

    def body(rows_ref, qw_ref, w1_ref, b1_ref, y_ref, ys_ref):
        @pl.when(pl.program_id(0) == 0)
        def _():
            ys_ref[...] = jnp.zeros_like(ys_ref)

        rows = rows_ref[...].astype(_f32).reshape(_TN_I, 8, CD)
        qwv = qw_ref[...]
        acc = rows[:, 0, :] * qwv[:, 0:1]
        for c in range(1, 8):
            acc = acc + rows[:, c, :] * qwv[:, c : c + 1]
        y = _dot(acc, w1_ref[...].T) + b1_ref[0:1, :]
        y_ref[...] = y
        ys_ref[0:1, :] += jnp.sum(y, axis=0, keepdims=True)
        ys_ref[1:2, :] += jnp.sum(y * y, axis=0, keepdims=True)

    return pl.pallas_call(
        body,
        grid=(nsteps,),
        in_specs=[
            pl.BlockSpec((_TN_I * 8, CD), lambda i: (i, 0)),
            pl.BlockSpec((_TN_I, 8), lambda i: (i, 0)),
            pl.BlockSpec((CD, CD), lambda i: (0, 0)),
            pl.BlockSpec((1, CD), lambda i: (0, 0)),
        ],
        out_specs=[
            pl.BlockSpec((_TN_I, CD), lambda i: (i, 0)),
            pl.BlockSpec((2, CD), lambda i: (0, 0)),
        ],
        out_shape=[
            jax.ShapeDtypeStruct((B * N, CD), _f32),
            jax.ShapeDtypeStruct((2, CD), _f32),
        ],
    )(qrows, qw, w1, b1)


_TN_A = 64


def _pass_a(crows, wcorn, pfx, p_exp, pdstats, wpd, ppd, w1, b1):
    nsteps = N // _TN_A
    rows = _TN_A * K

    def body(cr_ref, wc_ref, pf_ref, p_ref, st_ref, wpd_ref,
             ppd_ref, w1a_ref, w1b_ref, w1c_ref, b1_ref, y_ref, ys_ref):
        b = pl.program_id(0)
        step = pl.program_id(1)

        @pl.when(jnp.logical_and(b == 0, step == 0))
        def _():
            ys_ref[...] = jnp.zeros_like(ys_ref)

        cr = cr_ref[...].astype(_f32).reshape(rows, 8, CD)
        wc = wc_ref[...]
        vfeat = cr[:, 0, :] * wc[:, 0:1]
        for c in range(1, 8):
            vfeat = vfeat + cr[:, c, :] * wc[:, c : c + 1]

        pd = p_ref[...] - pf_ref[:, CD : CD + 3]
        wpd = wpd_ref[...]
        ypd = _dot(pd, wpd.T) + ppd_ref[0:1, :]
        n = float(B * NK)
        st = st_ref[...]
        mx, my, mz = st[0, 0] / n, st[0, 1] / n, st[0, 2] / n
        cxx = st[0, 3] / n - mx * mx
        cyy = st[0, 4] / n - my * my
        czz = st[0, 5] / n - mz * mz
        cxy = st[0, 6] / n - mx * my
        cxz = st[0, 7] / n - mx * mz
        cyz = st[0, 8] / n - my * mz
        wx, wy, wz = wpd[:, 0:1].T, wpd[:, 1:2].T, wpd[:, 2:3].T
        mu = wx * mx + wy * my + wz * mz + ppd_ref[0:1, :]
        var = (
            wx * wx * cxx + wy * wy * cyy + wz * wz * czz
            + 2.0 * (wx * wy * cxy + wx * wz * cxz + wy * wz * cyz)
        )
        sc = ppd_ref[1:2, :] * lax.rsqrt(var + EPS)
        sh = ppd_ref[2:3, :] - mu * sc
        xpd = _leaky(ypd * sc + sh)

        y1 = (
            _dot(pf_ref[:, 0:CD], w1a_ref[...].T)
            + _dot(vfeat, w1b_ref[...].T)
            + _dot(xpd, w1c_ref[...].T)
            + b1_ref[0:1, :]
        )
        y_ref[...] = y1
        ys_ref[0:1, :] += jnp.sum(y1, axis=0, keepdims=True)
        ys_ref[1:2, :] += jnp.sum(y1 * y1, axis=0, keepdims=True)

    return pl.pallas_call(
        body,
        grid=(B, nsteps),
        in_specs=[
            pl.BlockSpec((rows * 8, CD), lambda b, i: (b * nsteps + i, 0)),
            pl.BlockSpec((rows, 8), lambda b, i: (b * nsteps + i, 0)),
            pl.BlockSpec((rows, 2 * CD), lambda b, i: (b * nsteps + i, 0)),
            pl.BlockSpec((rows, 3), lambda b, i: (b * nsteps + i, 0)),
            pl.BlockSpec((1, 16), lambda b, i: (0, 0)),
            pl.BlockSpec((CD, 3), lambda b, i: (0, 0)),
            pl.BlockSpec((3, CD), lambda b, i: (0, 0)),
            pl.BlockSpec((CD, CD), lambda b, i: (0, 0)),
            pl.BlockSpec((CD, CD), lambda b, i: (0, 0)),
            pl.BlockSpec((CD, CD), lambda b, i: (0, 0)),
            pl.BlockSpec((1, CD), lambda b, i: (0, 0)),
        ],
        out_specs=[
            pl.BlockSpec((rows, CD), lambda b, i: (b * nsteps + i, 0)),
            pl.BlockSpec((2, CD), lambda b, i: (0, 0)),
        ],
        out_shape=[
            jax.ShapeDtypeStruct((B * NK, CD), _f32),
            jax.ShapeDtypeStruct((2, CD), _f32),
        ],
    )(crows, wcorn, pfx, p_exp, pdstats,
      wpd, ppd, w1[:, 0:CD], w1[:, CD : 2 * CD], w1[:, 2 * CD :], b1)


_TROW = 2048


def _pass_bn_mm(y_in, ystats, gprev, beprev, w, bnext, nrows, trow=_TROW):
    nsteps = nrows // trow

    def body(y_ref, ys_ref, gb_ref, w_ref, o_ref, os_ref):
        @pl.when(pl.program_id(0) == 0)
        def _():
            os_ref[...] = jnp.zeros_like(os_ref)

        n = float(nrows)
        ys = ys_ref[...]
        mu = ys[0:1, :] / n
        var = ys[1:2, :] / n - mu * mu
        sc = gb_ref[0:1, :] * lax.rsqrt(var + EPS)
        sh = gb_ref[1:2, :] - mu * sc
        x = _leaky(y_ref[...] * sc + sh)
        y2 = _dot(x, w_ref[...].T) + gb_ref[2:3, :]
        o_ref[...] = y2
        os_ref[0:1, :] += jnp.sum(y2, axis=0, keepdims=True)
        os_ref[1:2, :] += jnp.sum(y2 * y2, axis=0, keepdims=True)

    gb = jnp.concatenate(
        [gprev[None, :], beprev[None, :], bnext[None, :]], axis=0
    )
    return pl.pallas_call(
        body,
        grid=(nsteps,),
        in_specs=[
            pl.BlockSpec((trow, CD), lambda i: (i, 0)),
            pl.BlockSpec((2, CD), lambda i: (0, 0)),
            pl.BlockSpec((3, CD), lambda i: (0, 0)),
            pl.BlockSpec((CD, CD), lambda i: (0, 0)),
        ],
        out_specs=[
            pl.BlockSpec((trow, CD), lambda i: (i, 0)),
            pl.BlockSpec((2, CD), lambda i: (0, 0)),
        ],
        out_shape=[
            jax.ShapeDtypeStruct((nrows, CD), _f32),
            jax.ShapeDtypeStruct((2, CD), _f32),
        ],
    )(y_in, ystats, gb, w)


_TN_D = 256


def _pass_d(y2, y2s, y3, y3s, g2, be2, g3, be3, wkw2, bkw2,
            yi2, yi2s, gi2, bei2, wb1, bb1):
    nsteps = N // _TN_D
    rows = _TN_D * K

    def body(y2_ref, y2s_ref, y3_ref, y3s_ref, pv_ref, kw2_ref,
             yi2_ref, yi2s_ref, wb1a_ref, wb1b_ref, y4_ref, y4s_ref):
        b = pl.program_id(0)
        step = pl.program_id(1)

        @pl.when(jnp.logical_and(b == 0, step == 0))
        def _():
            y4s_ref[...] = jnp.zeros_like(y4s_ref)

        n = float(B * NK)
        y2s = y2s_ref[...]
        mu2 = y2s[0:1, :] / n
        var2 = y2s[1:2, :] / n - mu2 * mu2
        sc2 = pv_ref[0:1, :] * lax.rsqrt(var2 + EPS)
        sh2 = pv_ref[1:2, :] - mu2 * sc2
        x2 = _leaky(y2_ref[...] * sc2 + sh2)

        y3s = y3s_ref[...]
        mu3 = y3s[0:1, :] / n
        var3 = y3s[1:2, :] / n - mu3 * mu3
        sc3 = pv_ref[2:3, :] * lax.rsqrt(var3 + EPS)
        sh3 = pv_ref[3:4, :] - mu3 * sc3
        x3 = _leaky(y3_ref[...] * sc3 + sh3)

        s = jnp.sum(x3 * kw2_ref[0:1, :], axis=1, keepdims=True)
        sg = s.reshape(_TN_D, K, 1)
        smax = sg[:, 0, :]
        for k in range(1, K):
            smax = jnp.maximum(smax, sg[:, k, :])
        x2g = x2.reshape(_TN_D, K, CD)
        den = jnp.zeros((_TN_D, 1), _f32)
        vf = jnp.zeros((_TN_D, CD), _f32)
        for k in range(K):
            e = jnp.exp(sg[:, k, :] - smax)
            den = den + e
            vf = vf + x2g[:, k, :] * e
        vf = vf / den

        ni = float(B * N)
        yis = yi2s_ref[...]
        mui = yis[0:1, :] / ni
        vari = yis[1:2, :] / ni - mui * mui
        sci = pv_ref[6:7, :] * lax.rsqrt(vari + EPS)
        shi = pv_ref[7:8, :] - mui * sci
        vfp = _leaky(yi2_ref[...] * sci + shi)

        y4 = (
            _dot(vf, wb1a_ref[...].T)
            + _dot(vfp, wb1b_ref[...].T)
            + pv_ref[5:6, :]
        )
        y4_ref[...] = y4
        y4s_ref[0:1, :] += jnp.sum(y4, axis=0, keepdims=True)
        y4s_ref[1:2, :] += jnp.sum(y4 * y4, axis=0, keepdims=True)

    pvec = jnp.stack(
        [g2, be2, g3, be3,
         jnp.full((CD,), bkw2[0], _f32), bb1, gi2, bei2], axis=0
    )
    return pl.pallas_call(
        body,
        grid=(B, nsteps),
        in_specs=[
            pl.BlockSpec((rows, CD), lambda b, i: (b * nsteps + i, 0)),
            pl.BlockSpec((2, CD), lambda b, i: (0, 0)),
            pl.BlockSpec((rows, CD), lambda b, i: (b * nsteps + i, 0)),
            pl.BlockSpec((2, CD), lambda b, i: (0, 0)),
            pl.BlockSpec((8, CD), lambda b, i: (0, 0)),
            pl.BlockSpec((1, CD), lambda b, i: (0, 0)),
            pl.BlockSpec((_TN_D, CD), lambda b, i: (b * nsteps + i, 0)),
            pl.BlockSpec((2, CD), lambda b, i: (0, 0)),
            pl.BlockSpec((CD, CD), lambda b, i: (0, 0)),
            pl.BlockSpec((CD, CD), lambda b, i: (0, 0)),
        ],
        out_specs=[
            pl.BlockSpec((_TN_D, CD), lambda b, i: (b * nsteps + i, 0)),
            pl.BlockSpec((2, CD), lambda b, i: (0, 0)),
        ],
        out_shape=[
            jax.ShapeDtypeStruct((B * N, CD), _f32),
            jax.ShapeDtypeStruct((2, CD), _f32),
        ],
    )(y2, y2s, y3, y3s, pvec, wkw2, yi2, yi2s, wb1[:, 0:CD], wb1[:, CD:])


def _final(y4, y4s, g, be, wb2, bb2):
    def body(y_ref, ys_ref, pv_ref, w_ref, o_ref):
        n = float(B * N)
        ys = ys_ref[...]
        mu = ys[0:1, :] / n
        var = ys[1:2, :] / n - mu * mu
        sc = pv_ref[0:1, :] * lax.rsqrt(var + EPS)
        sh = pv_ref[1:2, :] - mu * sc
        x = _leaky(y_ref[...] * sc + sh)
        o_ref[...] = (
            jnp.sum(x * w_ref[0:1, :], axis=1, keepdims=True) + pv_ref[2, 0]
        )

    pvec = jnp.stack([g, be, jnp.full((CD,), bb2[0], _f32)], axis=0)
    return pl.pallas_call(
        body,
        grid=(1,),
        in_specs=[
            pl.BlockSpec((B * N, CD), lambda i: (0, 0)),
            pl.BlockSpec((2, CD), lambda i: (0, 0)),
            pl.BlockSpec((3, CD), lambda i: (0, 0)),
            pl.BlockSpec((1, CD), lambda i: (0, 0)),
        ],
        out_specs=pl.BlockSpec((B * N, 1), lambda i: (0, 0)),
        out_shape=jax.ShapeDtypeStruct((B * N, 1), _f32),
    )(y4, y4s, pvec, wb2)


def kernel(p, z, feat_g, feat_out, feat_pt, inputs, params):
    pr = params

    gt = _grid_transpose(feat_out)
    fptt = _point_cba(
        feat_pt, inputs,
        pr["point_W"], pr["point_b"], pr["point_g"], pr["point_be"]
    )
    qcidx, qw = _prep(p)
    idxs = _knn(p, inputs)

    idx_flat = idxs.reshape(B * NK)
    p_exp = jnp.broadcast_to(p[:, :, None, :], (B, N, K, 3)).reshape(B * NK, 3)

    qrows = _gather_rows(gt, qcidx.reshape(B * N * 8), 512)
    pfx = _gather_rows(fptt, idx_flat, 256)

    cidx, wcorn, pdstats = _extract(pfx, p_exp)
    crows = _gather_rows(gt, cidx.reshape(B * NK * 8), 512)

    yi1, yi1s = _interp1(qrows, qw, pr["interp1_W"], pr["interp1_b"][None, :])
    yi2, yi2s = _pass_bn_mm(
        yi1, yi1s, pr["interp1_g"], pr["interp1_be"],
        pr["interp2_W"], pr["interp2_b"], B * N, 1024,
    )

    y1, y1s = _pass_a(
        crows, wcorn, pfx, p_exp, pdstats,
        pr["pdiff_W"],
        jnp.stack([pr["pdiff_b"], pr["pdiff_g"], pr["pdiff_be"]], 0),
        pr["knn1_W"], pr["knn1_b"][None, :],
    )
    y2, y2s = _pass_bn_mm(
        y1, y1s, pr["knn1_g"], pr["knn1_be"], pr["knn2_W"], pr["knn2_b"],
        B * NK,
    )
    y3, y3s = _pass_bn_mm(
        y2, y2s, pr["knn2_g"], pr["knn2_be"], pr["kw1_W"], pr["kw1_b"],
        B * NK,
    )
    y4, y4s = _pass_d(
        y2, y2s, y3, y3s,
        pr["knn2_g"], pr["knn2_be"], pr["kw1_g"], pr["kw1_be"],
        pr["kw2_W"], pr["kw2_b"],
        yi2, yi2s, pr["interp2_g"], pr["interp2_be"],
        pr["block1_W"], pr["block1_b"],
    )
    out = _final(y4, y4s, pr["block1_g"], pr["block1_be"],
                 pr["block2_W"], pr["block2_b"])
    return out.reshape(B, 1, N)
